# baseline (speedup 1.0000x reference)
import functools

import jax
import jax.numpy as jnp
from jax import lax
from jax.experimental import pallas as pl
from jax.experimental.pallas import tpu as pltpu
from jax.experimental.pallas import tpu_sc as plsc

TABLE_ROWS = 8192
TABLE_COLS = 8192
N_LOOKUPS = 1048576

NC = 2
NS = 16
NW = NC * NS
L = 16

N_PER_W = N_LOOKUPS // NW
CHUNK = 4096
N_SUPER = N_PER_W // CHUNK
G = 128
K = CHUNK // G


def _body(idx_hbm, tab_hbm, out_hbm, idx_stage, fidx, outbuf, sem):
    wid = lax.axis_index("s") * NC + lax.axis_index("c")
    lane = lax.iota(jnp.int32, L)

    def super_chunk(s, _):
        base = wid * N_PER_W + s * CHUNK
        pltpu.sync_copy(idx_hbm.at[pl.ds(base * 2, 2 * CHUNK)], idx_stage)

        def fcomp(j, _):
            ev = lane * 2 + j * (2 * L)
            r = plsc.load_gather(idx_stage, [ev])
            c = plsc.load_gather(idx_stage, [ev + 1])
            fidx[pl.ds(j * L, L)] = r * TABLE_COLS + c
            return 0

        lax.fori_loop(0, CHUNK // L, fcomp, 0)

        def fire(k, _):
            pltpu.async_copy(
                tab_hbm.at[fidx.at[pl.ds(k * G, G)]],
                outbuf.at[pl.ds(k * G, G)],
                sem,
            )
            return 0

        lax.fori_loop(0, K, fire, 0)
        pltpu.make_async_copy(tab_hbm.at[pl.ds(0, CHUNK)], outbuf, sem).wait()

        pltpu.sync_copy(outbuf, out_hbm.at[pl.ds(base, CHUNK)])
        return 0

    lax.fori_loop(0, N_SUPER, super_chunk, 0)


@jax.jit
def _run(idx_flat, tab_flat):
    mesh = plsc.VectorSubcoreMesh(core_axis_name="c", subcore_axis_name="s")
    f = functools.partial(
        pl.kernel,
        mesh=mesh,
        out_type=jax.ShapeDtypeStruct((N_LOOKUPS,), jnp.float32),
        scratch_types=[
            pltpu.VMEM((2 * CHUNK,), jnp.int32),
            pltpu.VMEM((CHUNK,), jnp.int32),
            pltpu.VMEM((CHUNK,), jnp.float32),
            pltpu.SemaphoreType.DMA,
        ],
        compiler_params=pltpu.CompilerParams(needs_layout_passes=False),
    )(_body)
    return f(idx_flat, tab_flat)


def kernel(indices, train_table):
    idx_flat = indices.astype(jnp.int32).reshape(-1)
    tab_flat = train_table.reshape(-1)
    return _run(idx_flat, tab_flat)


# speedup vs baseline: 1.0729x; 1.0729x over previous
import functools

import jax
import jax.numpy as jnp
from jax import lax
from jax.experimental import pallas as pl
from jax.experimental.pallas import tpu as pltpu
from jax.experimental.pallas import tpu_sc as plsc

TABLE_ROWS = 8192
TABLE_COLS = 8192
N_LOOKUPS = 1048576

NC = 2
NS = 16
NW = NC * NS
L = 16

N_PER_W = N_LOOKUPS // NW
CHUNK = 4096
N_SUPER = N_PER_W // CHUNK
G = 128
K = CHUNK // G


def _body(idx_hbm, tab_hbm, out_hbm, idx_stage, fidx, outbuf, sem):
    wid = lax.axis_index("s") * NC + lax.axis_index("c")
    lane = lax.iota(jnp.int32, L)

    def super_chunk(s, _):
        base = wid * N_PER_W + s * CHUNK
        pltpu.sync_copy(idx_hbm.at[pl.ds(base * 2, 2 * CHUNK)], idx_stage)

        def fcomp(j, _):
            ev = lane * 2 + j * (2 * L)
            r = plsc.load_gather(idx_stage, [ev])
            c = plsc.load_gather(idx_stage, [ev + 1])
            phys = (
                ((r >> 3) << 16)
                + ((c >> 7) << 10)
                + ((r & 7) << 7)
                + (c & 127)
            )
            fidx[pl.ds(j * L, L)] = phys
            return 0

        lax.fori_loop(0, CHUNK // L, fcomp, 0)

        def fire(k, _):
            pltpu.async_copy(
                tab_hbm.at[fidx.at[pl.ds(k * G, G)]],
                outbuf.at[pl.ds(k * G, G)],
                sem,
            )
            return 0

        lax.fori_loop(0, K, fire, 0)
        pltpu.make_async_copy(tab_hbm.at[pl.ds(0, CHUNK)], outbuf, sem).wait()

        pltpu.sync_copy(outbuf, out_hbm.at[pl.ds(base, CHUNK)])
        return 0

    lax.fori_loop(0, N_SUPER, super_chunk, 0)


@jax.jit
def _run(idx_flat, tab_lin):
    mesh = plsc.VectorSubcoreMesh(core_axis_name="c", subcore_axis_name="s")
    f = functools.partial(
        pl.kernel,
        mesh=mesh,
        out_type=jax.ShapeDtypeStruct((N_LOOKUPS,), jnp.float32),
        scratch_types=[
            pltpu.VMEM((2 * CHUNK,), jnp.int32),
            pltpu.VMEM((CHUNK,), jnp.int32),
            pltpu.VMEM((CHUNK,), jnp.float32),
            pltpu.SemaphoreType.DMA,
        ],
        compiler_params=pltpu.CompilerParams(needs_layout_passes=False),
    )(_body)
    return f(idx_flat, tab_lin)


def kernel(indices, train_table):
    idx_flat = indices.astype(jnp.int32).reshape(-1)
    tab_lin = (
        train_table.reshape(1024, 8, 64, 128)
        .transpose(0, 2, 1, 3)
        .reshape(TABLE_ROWS * TABLE_COLS)
    )
    return _run(idx_flat, tab_lin)
